# Initial kernel scaffold; baseline (speedup 1.0000x reference)
import jax
import jax.numpy as jnp
from jax.experimental import pallas as pl


def kernel(feats):
    raise NotImplementedError("write your pallas kernel here")


# speedup vs baseline: 5.6586x; 5.6586x over previous
import functools

import jax
import jax.numpy as jnp
from jax import lax
from jax.experimental import pallas as pl
from jax.experimental.pallas import tpu as pltpu
from jax.experimental.pallas import tpu_sc as plsc

B = 8
T = 4096
D = 512
TT = 4095
NT = 1365
RB = NT * 5
NW = 32
K = 96
PB = 6912
CPB = PB // K
NCHUNK = B * CPB
PAD_ROWS = B * PB

_mesh = plsc.VectorSubcoreMesh(
    core_axis_name="c", subcore_axis_name="s", num_cores=2, num_subcores=16
)


@functools.partial(
    pl.kernel,
    mesh=_mesh,
    out_type=jax.ShapeDtypeStruct((PAD_ROWS, D), jnp.float32),
    scratch_types=[
        pltpu.VMEM((K,), jnp.int32),
        pltpu.VMEM((K, D), jnp.float32),
        pltpu.SemaphoreType.DMA,
    ],
)
def _splice_gather(feats_hbm, out_hbm, idx_v, rows_v, sem):
    wid = lax.axis_index("s") * 2 + lax.axis_index("c")
    lanes = lax.iota(jnp.int32, 16)

    @pl.loop(0, NCHUNK // NW)
    def _chunk(j):
        cc = wid * (NCHUNK // NW) + j
        b = (cc * 911) >> 16
        q = cc - b * CPB
        r0 = q * K
        for i in range(K // 16):
            r = r0 + i * 16 + lanes
            t = (r * 52429) >> 18
            k = r - t * 5
            src = jnp.clip(3 * t + k - 2, 0, TT - 1)
            idx_v[pl.ds(i * 16, 16)] = src + b * T
        pltpu.async_copy(feats_hbm.at[idx_v], rows_v, sem).wait()
        pltpu.sync_copy(rows_v, out_hbm.at[pl.ds(b * PB + r0, K)])


def kernel(feats):
    flat = feats.reshape(B * T, D)
    out = _splice_gather(flat)
    return out.reshape(B, PB, D)[:, :RB].reshape(B, NT, 5 * D)


# speedup vs baseline: 5.9365x; 1.0491x over previous
import functools

import jax
import jax.numpy as jnp
from jax import lax
from jax.experimental import pallas as pl
from jax.experimental.pallas import tpu as pltpu
from jax.experimental.pallas import tpu_sc as plsc

B = 8
T = 4096
D = 512
TT = 4095
NT = 1365
RB = NT * 5
NW = 32
K = 96
PB = 6912
CPB = PB // K
NCHUNK = B * CPB
PAD_ROWS = B * PB

_mesh = plsc.VectorSubcoreMesh(
    core_axis_name="c", subcore_axis_name="s", num_cores=2, num_subcores=16
)


CPW = NCHUNK // NW


@functools.partial(
    pl.kernel,
    mesh=_mesh,
    out_type=jax.ShapeDtypeStruct((PAD_ROWS, D), jnp.float32),
    scratch_types=[
        pltpu.VMEM((K,), jnp.int32),
        pltpu.VMEM((K,), jnp.int32),
        pltpu.VMEM((K, D), jnp.float32),
        pltpu.VMEM((K, D), jnp.float32),
        pltpu.SemaphoreType.DMA,
        pltpu.SemaphoreType.DMA,
        pltpu.SemaphoreType.DMA,
        pltpu.SemaphoreType.DMA,
    ],
)
def _splice_gather(feats_hbm, out_hbm, idx0_v, idx1_v, rows0_v, rows1_v,
                   sem_g0, sem_g1, sem_o0, sem_o1):
    wid = lax.axis_index("s") * 2 + lax.axis_index("c")
    lanes = lax.iota(jnp.int32, 16)

    def decomp(cc):
        b = (cc * 911) >> 16
        q = cc - b * CPB
        return b, q * K

    def fill_idx(idx_v, cc):
        b, r0 = decomp(cc)
        for i in range(K // 16):
            r = r0 + i * 16 + lanes
            t = (r * 52429) >> 18
            k = r - t * 5
            src = jnp.clip(3 * t + k - 2, 0, TT - 1)
            idx_v[pl.ds(i * 16, 16)] = src + b * T

    def out_ref(cc):
        b, r0 = decomp(cc)
        return out_hbm.at[pl.ds(b * PB + r0, K)]

    def start_gather(idx_v, rows_v, sem, cc):
        fill_idx(idx_v, cc)
        pltpu.async_copy(feats_hbm.at[idx_v], rows_v, sem)

    def wait_gather(idx_v, rows_v, sem):
        pltpu.make_async_copy(feats_hbm.at[idx_v], rows_v, sem).wait()

    def start_put(rows_v, sem, cc):
        pltpu.async_copy(rows_v, out_ref(cc), sem)

    def wait_put(rows_v, sem, cc):
        pltpu.make_async_copy(rows_v, out_ref(cc), sem).wait()

    c0 = wid * CPW
    start_gather(idx0_v, rows0_v, sem_g0, c0)

    @pl.loop(0, CPW // 2)
    def _pair(m):
        c = c0 + 2 * m

        @pl.when(m > 0)
        def _():
            wait_put(rows1_v, sem_o1, c - 1)

        start_gather(idx1_v, rows1_v, sem_g1, c + 1)
        wait_gather(idx0_v, rows0_v, sem_g0)
        start_put(rows0_v, sem_o0, c)

        @pl.when(m < CPW // 2 - 1)
        def _():
            wait_put(rows0_v, sem_o0, c)
            start_gather(idx0_v, rows0_v, sem_g0, c + 2)

        wait_gather(idx1_v, rows1_v, sem_g1)
        start_put(rows1_v, sem_o1, c + 1)

    wait_put(rows0_v, sem_o0, c0 + CPW - 2)
    wait_put(rows1_v, sem_o1, c0 + CPW - 1)


def kernel(feats):
    flat = feats.reshape(B * T, D)
    out = _splice_gather(flat)
    return out.reshape(B, PB, D)[:, :RB].reshape(B, NT, 5 * D)
